# baseline (speedup 1.0000x reference)
import functools
import math

import jax
import jax.numpy as jnp
import numpy as _np
from jax import lax
from jax.experimental import pallas as pl
from jax.experimental.pallas import tpu as pltpu
from jax.experimental.pallas import tpu_sc as plsc

D = H = W = 256
N = 2097152
DHW = D * H * W

NC = 2
NS = 16
L = 16
NW = NC * NS
NPW = N // NW
B = 1024
GROUPS = B // L
CHUNKS = NPW // B
GCHUNK = 128
SUBB = B // GCHUNK
GPS = GCHUNK // L
GROWS = 8 * SUBB

_T = float(_np.float32(0.001))
_C = -2.0 * math.log1p(-_T)
_D_THRES = 4.0 + math.log(math.expm1(_C))

_mesh = plsc.VectorSubcoreMesh(
    core_axis_name="c", subcore_axis_name="s", num_cores=NC, num_subcores=NS)

_DNUMS = lax.GatherDimensionNumbers(
    offset_dims=(), collapsed_slice_dims=(0,), start_index_map=(0,))


def _shuf(v, sidx):
    return lax.gather(v, sidx.reshape(L, 1), _DNUMS, (1,),
                      mode=lax.GatherScatterMode.PROMISE_IN_BOUNDS)


@functools.partial(
    pl.kernel,
    out_type=jax.ShapeDtypeStruct((N,), jnp.int32),
    mesh=_mesh,
    scratch_types=[
        pltpu.VMEM((2, 3 * B), jnp.float32),
        pltpu.VMEM((2, B), jnp.float32),
        pltpu.VMEM((2, B), jnp.float32),
        pltpu.VMEM((2, B), jnp.float32),
        pltpu.VMEM((2, GROWS, GCHUNK), jnp.int32),
        pltpu.VMEM((2, GROWS, GCHUNK), jnp.float32),
        pltpu.VMEM((B,), jnp.int32),
        pltpu.VMEM((6, L), jnp.float32),
        pltpu.SemaphoreType.DMA,
        pltpu.SemaphoreType.DMA,
        pltpu.SemaphoreType.DMA,
    ],
)
def _sc_kernel(xyz3_hbm, dens_hbm, dens2d_hbm, sc_hbm, out_hbm,
               cxyz, wx, wy, wz, idx, vals, mask, scales,
               sem_g0, sem_g1, sem_c):
    wid = lax.axis_index("s") * NC + lax.axis_index("c")
    base = wid * NPW

    pltpu.sync_copy(sc_hbm, scales)
    sW = scales[0, :]
    sH = scales[1, :]
    sD = scales[2, :]
    oW = scales[3, :]
    oH = scales[4, :]
    oD = scales[5, :]
    thres = jnp.full((L,), _D_THRES, dtype=jnp.float32)
    one = jnp.full((L,), 1, jnp.int32)
    zero = jnp.full((L,), 0, jnp.int32)

    io = lax.broadcasted_iota(jnp.int32, (L,), 0)
    dss = []
    for off in range(3):
        fl = io * 3 + off
        dss.append((fl & 15, (fl - 16) & 15, (fl - 32) & 15,
                    fl < 16, fl < 32))

    def deint(v0, v1, v2, t):
        s0, s1, s2, m0, m1 = t
        return jnp.where(m0, _shuf(v0, s0),
                         jnp.where(m1, _shuf(v1, s1), _shuf(v2, s2)))

    pltpu.async_copy(xyz3_hbm.at[pl.ds(base * 3, 3 * B)], cxyz.at[0], sem_c)

    def gen(c, p, sem_g):
        def gen_sub(s, _):
            r0 = s * 8
            for j in range(GPS):
                i = s * GPS + j
                fbase = i * (3 * L)
                v0 = cxyz[p, pl.ds(fbase, L)]
                v1 = cxyz[p, pl.ds(fbase + L, L)]
                v2 = cxyz[p, pl.ds(fbase + 2 * L, L)]
                xs = deint(v0, v1, v2, dss[0])
                ys = deint(v0, v1, v2, dss[1])
                zs = deint(v0, v1, v2, dss[2])
                fW = zs * sW + oW
                fH = ys * sH + oH
                fD = xs * sD + oD
                x0 = jnp.clip(fW.astype(jnp.int32), 0, W - 2)
                y0 = jnp.clip(fH.astype(jnp.int32), 0, H - 2)
                z0 = jnp.clip(fD.astype(jnp.int32), 0, D - 2)
                wx[p, pl.ds(i * L, L)] = fW - x0.astype(jnp.float32)
                wy[p, pl.ds(i * L, L)] = fH - y0.astype(jnp.float32)
                wz[p, pl.ds(i * L, L)] = fD - z0.astype(jnp.float32)
                i000 = (z0 << 16) | (y0 << 8) | x0
                cc = j * L
                idx[p, r0 + 0, pl.ds(cc, L)] = i000
                idx[p, r0 + 1, pl.ds(cc, L)] = i000 + 1
                idx[p, r0 + 2, pl.ds(cc, L)] = i000 + W
                idx[p, r0 + 3, pl.ds(cc, L)] = i000 + (W + 1)
                idx[p, r0 + 4, pl.ds(cc, L)] = i000 + H * W
                idx[p, r0 + 5, pl.ds(cc, L)] = i000 + (H * W + 1)
                idx[p, r0 + 6, pl.ds(cc, L)] = i000 + (H * W + W)
                idx[p, r0 + 7, pl.ds(cc, L)] = i000 + (H * W + W + 1)
            for k in range(8):
                pltpu.async_copy(dens_hbm.at[idx.at[p, r0 + k]],
                                 vals.at[p, r0 + k], sem_g)
            return 0

        lax.fori_loop(0, SUBB, gen_sub, 0)

    def combine_out(c, p, sem_g):
        pltpu.make_async_copy(dens2d_hbm.at[pl.ds(0, GROWS), :],
                              vals.at[p], sem_g).wait()

        def combine(i, _):
            s = i * L
            r0 = (i >> 3) * 8
            cc = (i & 7) * L
            g000 = vals[p, r0 + 0, pl.ds(cc, L)]
            g001 = vals[p, r0 + 1, pl.ds(cc, L)]
            g010 = vals[p, r0 + 2, pl.ds(cc, L)]
            g011 = vals[p, r0 + 3, pl.ds(cc, L)]
            g100 = vals[p, r0 + 4, pl.ds(cc, L)]
            g101 = vals[p, r0 + 5, pl.ds(cc, L)]
            g110 = vals[p, r0 + 6, pl.ds(cc, L)]
            g111 = vals[p, r0 + 7, pl.ds(cc, L)]
            ax = wx[p, pl.ds(s, L)]
            ay = wy[p, pl.ds(s, L)]
            az = wz[p, pl.ds(s, L)]
            c00 = g000 + (g001 - g000) * ax
            c01 = g010 + (g011 - g010) * ax
            c10 = g100 + (g101 - g100) * ax
            c11 = g110 + (g111 - g110) * ax
            c0 = c00 + (c01 - c00) * ay
            c1 = c10 + (c11 - c10) * ay
            d = c0 + (c1 - c0) * az
            mask[pl.ds(s, L)] = jnp.where(d >= thres, one, zero)
            return 0

        lax.fori_loop(0, GROUPS, combine, 0)
        pltpu.sync_copy(mask, out_hbm.at[pl.ds(base + c * B, B)])

    def half(c, p):
        q = 1 - p
        semp = sem_g0 if p == 0 else sem_g1
        semq = sem_g1 if p == 0 else sem_g0
        pltpu.make_async_copy(xyz3_hbm.at[pl.ds(0, 3 * B)],
                              cxyz.at[p], sem_c).wait()
        cn = jnp.minimum(c + 1, CHUNKS - 1)
        pltpu.async_copy(xyz3_hbm.at[pl.ds((base + cn * B) * 3, 3 * B)],
                         cxyz.at[q], sem_c)
        gen(c, p, semp)
        pl.when(c > 0)(lambda: combine_out(c - 1, q, semq))

    def pair_body(t, _):
        half(2 * t, 0)
        half(2 * t + 1, 1)
        return 0

    lax.fori_loop(0, CHUNKS // 2, pair_body, 0)
    combine_out(CHUNKS - 1, 1, sem_g1)
    pltpu.make_async_copy(xyz3_hbm.at[pl.ds(0, 3 * B)],
                          cxyz.at[0], sem_c).wait()


def kernel(xyz, density, xyz_min, xyz_max):
    xyz3 = xyz.reshape(-1)
    dens_flat = density.reshape(-1)
    inv = 255.0 / (xyz_max - xyz_min)
    s = inv[::-1]
    o = (-xyz_min * inv)[::-1]
    sc = jnp.broadcast_to(
        jnp.concatenate([s, o]).reshape(6, 1).astype(jnp.float32), (6, L))
    dens2d = density.reshape(DHW // GCHUNK, GCHUNK)
    m = _sc_kernel(xyz3, dens_flat, dens2d, sc)
    return m.astype(jnp.bool_)


# speedup vs baseline: 4.3839x; 4.3839x over previous
import functools
import math

import jax
import jax.numpy as jnp
import numpy as _np
from jax import lax
from jax.experimental import pallas as pl
from jax.experimental.pallas import tpu as pltpu
from jax.experimental.pallas import tpu_sc as plsc

D = H = W = 256
N = 2097152
DHW = D * H * W

NC = 2
NS = 16
L = 16
NW = NC * NS
NPW = N // NW
B = 1024
GROUPS = B // L
CHUNKS = NPW // B
GCHUNK = 128
SUBB = B // GCHUNK
GPS = GCHUNK // L
GROWS = 8 * SUBB

_T = float(_np.float32(0.001))
_C = -2.0 * math.log1p(-_T)
_D_THRES = 4.0 + math.log(math.expm1(_C))

_mesh = plsc.VectorSubcoreMesh(
    core_axis_name="c", subcore_axis_name="s", num_cores=NC, num_subcores=NS)

_DNUMS = lax.GatherDimensionNumbers(
    offset_dims=(), collapsed_slice_dims=(0,), start_index_map=(0,))


def _shuf(v, sidx):
    return lax.gather(v, sidx.reshape(L, 1), _DNUMS, (1,),
                      mode=lax.GatherScatterMode.PROMISE_IN_BOUNDS)


@functools.partial(
    pl.kernel,
    out_type=jax.ShapeDtypeStruct((N,), jnp.int32),
    mesh=_mesh,
    scratch_types=[
        pltpu.VMEM((2, B), jnp.float32),
        pltpu.VMEM((2, B), jnp.float32),
        pltpu.VMEM((2, B), jnp.float32),
        pltpu.VMEM((2, B), jnp.float32),
        pltpu.VMEM((2, B), jnp.float32),
        pltpu.VMEM((2, B), jnp.float32),
        pltpu.VMEM((2, GROWS, GCHUNK), jnp.int32),
        pltpu.VMEM((2, GROWS, GCHUNK), jnp.float32),
        pltpu.VMEM((B,), jnp.int32),
        pltpu.VMEM((6, L), jnp.float32),
        pltpu.SemaphoreType.DMA,
        pltpu.SemaphoreType.DMA,
        pltpu.SemaphoreType.DMA,
    ],
)
def _sc_kernel(xs_hbm, ys_hbm, zs_hbm, dens_hbm, dens2d_hbm, sc_hbm, out_hbm,
               cx, cy, cz, wx, wy, wz, idx, vals, mask, scales,
               sem_g0, sem_g1, sem_c):
    wid = lax.axis_index("s") * NC + lax.axis_index("c")
    base = wid * NPW

    pltpu.sync_copy(sc_hbm, scales)
    sW = scales[0, :]
    sH = scales[1, :]
    sD = scales[2, :]
    oW = scales[3, :]
    oH = scales[4, :]
    oD = scales[5, :]
    thres = jnp.full((L,), _D_THRES, dtype=jnp.float32)
    one = jnp.full((L,), 1, jnp.int32)
    zero = jnp.full((L,), 0, jnp.int32)

    pltpu.async_copy(xs_hbm.at[pl.ds(base, B)], cx.at[0], sem_c)
    pltpu.async_copy(ys_hbm.at[pl.ds(base, B)], cy.at[0], sem_c)
    pltpu.async_copy(zs_hbm.at[pl.ds(base, B)], cz.at[0], sem_c)

    def gen(c, p, sem_g):
        def gen_sub(s, _):
            r0 = s * 8
            for j in range(GPS):
                i = s * GPS + j
                xs = cx[p, pl.ds(i * L, L)]
                ys = cy[p, pl.ds(i * L, L)]
                zs = cz[p, pl.ds(i * L, L)]
                fW = zs * sW + oW
                fH = ys * sH + oH
                fD = xs * sD + oD
                x0 = jnp.clip(fW.astype(jnp.int32), 0, W - 2)
                y0 = jnp.clip(fH.astype(jnp.int32), 0, H - 2)
                z0 = jnp.clip(fD.astype(jnp.int32), 0, D - 2)
                wx[p, pl.ds(i * L, L)] = fW - x0.astype(jnp.float32)
                wy[p, pl.ds(i * L, L)] = fH - y0.astype(jnp.float32)
                wz[p, pl.ds(i * L, L)] = fD - z0.astype(jnp.float32)
                i000 = (z0 << 16) | (y0 << 8) | x0
                cc = j * L
                idx[p, r0 + 0, pl.ds(cc, L)] = i000
                idx[p, r0 + 1, pl.ds(cc, L)] = i000 + 1
                idx[p, r0 + 2, pl.ds(cc, L)] = i000 + W
                idx[p, r0 + 3, pl.ds(cc, L)] = i000 + (W + 1)
                idx[p, r0 + 4, pl.ds(cc, L)] = i000 + H * W
                idx[p, r0 + 5, pl.ds(cc, L)] = i000 + (H * W + 1)
                idx[p, r0 + 6, pl.ds(cc, L)] = i000 + (H * W + W)
                idx[p, r0 + 7, pl.ds(cc, L)] = i000 + (H * W + W + 1)
            for k in range(8):
                pltpu.async_copy(dens_hbm.at[idx.at[p, r0 + k]],
                                 vals.at[p, r0 + k], sem_g)
            return 0

        lax.fori_loop(0, SUBB, gen_sub, 0)

    def combine_out(c, p, sem_g):
        pltpu.make_async_copy(dens2d_hbm.at[pl.ds(0, GROWS), :],
                              vals.at[p], sem_g).wait()

        def combine(i, _):
            s = i * L
            r0 = (i >> 3) * 8
            cc = (i & 7) * L
            g000 = vals[p, r0 + 0, pl.ds(cc, L)]
            g001 = vals[p, r0 + 1, pl.ds(cc, L)]
            g010 = vals[p, r0 + 2, pl.ds(cc, L)]
            g011 = vals[p, r0 + 3, pl.ds(cc, L)]
            g100 = vals[p, r0 + 4, pl.ds(cc, L)]
            g101 = vals[p, r0 + 5, pl.ds(cc, L)]
            g110 = vals[p, r0 + 6, pl.ds(cc, L)]
            g111 = vals[p, r0 + 7, pl.ds(cc, L)]
            ax = wx[p, pl.ds(s, L)]
            ay = wy[p, pl.ds(s, L)]
            az = wz[p, pl.ds(s, L)]
            c00 = g000 + (g001 - g000) * ax
            c01 = g010 + (g011 - g010) * ax
            c10 = g100 + (g101 - g100) * ax
            c11 = g110 + (g111 - g110) * ax
            c0 = c00 + (c01 - c00) * ay
            c1 = c10 + (c11 - c10) * ay
            d = c0 + (c1 - c0) * az
            mask[pl.ds(s, L)] = jnp.where(d >= thres, one, zero)
            return 0

        lax.fori_loop(0, GROUPS, combine, 0)
        pltpu.sync_copy(mask, out_hbm.at[pl.ds(base + c * B, B)])

    def half(c, p):
        q = 1 - p
        semp = sem_g0 if p == 0 else sem_g1
        semq = sem_g1 if p == 0 else sem_g0
        pltpu.make_async_copy(xs_hbm.at[pl.ds(0, B)], cx.at[p], sem_c).wait()
        pltpu.make_async_copy(xs_hbm.at[pl.ds(0, B)], cy.at[p], sem_c).wait()
        pltpu.make_async_copy(xs_hbm.at[pl.ds(0, B)], cz.at[p], sem_c).wait()
        cn = jnp.minimum(c + 1, CHUNKS - 1)
        pn = base + cn * B
        pltpu.async_copy(xs_hbm.at[pl.ds(pn, B)], cx.at[q], sem_c)
        pltpu.async_copy(ys_hbm.at[pl.ds(pn, B)], cy.at[q], sem_c)
        pltpu.async_copy(zs_hbm.at[pl.ds(pn, B)], cz.at[q], sem_c)
        gen(c, p, semp)
        pl.when(c > 0)(lambda: combine_out(c - 1, q, semq))

    def pair_body(t, _):
        half(2 * t, 0)
        half(2 * t + 1, 1)
        return 0

    lax.fori_loop(0, CHUNKS // 2, pair_body, 0)
    combine_out(CHUNKS - 1, 1, sem_g1)
    pltpu.make_async_copy(xs_hbm.at[pl.ds(0, B)], cx.at[0], sem_c).wait()
    pltpu.make_async_copy(xs_hbm.at[pl.ds(0, B)], cy.at[0], sem_c).wait()
    pltpu.make_async_copy(xs_hbm.at[pl.ds(0, B)], cz.at[0], sem_c).wait()


def kernel(xyz, density, xyz_min, xyz_max):
    xs = xyz[:, 0]
    ys = xyz[:, 1]
    zs = xyz[:, 2]
    dens_flat = density.reshape(-1)
    inv = 255.0 / (xyz_max - xyz_min)
    s = inv[::-1]
    o = (-xyz_min * inv)[::-1]
    sc = jnp.broadcast_to(
        jnp.concatenate([s, o]).reshape(6, 1).astype(jnp.float32), (6, L))
    dens2d = density.reshape(DHW // GCHUNK, GCHUNK)
    m = _sc_kernel(xs, ys, zs, dens_flat, dens2d, sc)
    return m.astype(jnp.bool_)
